# baseline (speedup 1.0000x reference)
import functools

import jax
import jax.numpy as jnp
from jax import lax
from jax.experimental import pallas as pl
from jax.experimental.pallas import tpu as pltpu


def _gelu_exact(x):
    return 0.5 * x * (1.0 + lax.erf(x * 0.7071067811865476))


def _sab_kernel(x_ref, wma_ref, wm_ref, ws_ref, bma_ref, bm_ref, b2_ref,
                wr_ref, br_ref, o_ref, *, cin, cmid, rt, ngroups):
    gr = rt * cin
    ga = rt * cmid
    wma = wma_ref[...]
    wm = wm_ref[...]
    ws = ws_ref[...]
    for g in range(ngroups):
        xg = x_ref[g * gr:(g + 1) * gr, :].astype(jnp.bfloat16)
        gma = _gelu_exact(
            jnp.dot(wma, xg, preferred_element_type=jnp.float32)
            + bma_ref[...]).astype(jnp.bfloat16)
        hm = gma[:gr]
        ha = gma[gr:gr + ga]
        outm = jnp.dot(wm, hm, preferred_element_type=jnp.float32) + bm_ref[...]
        s = jax.nn.sigmoid(
            jnp.dot(ws, ha, preferred_element_type=jnp.float32) + b2_ref[...])
        r = s * wr_ref[...] + br_ref[...]
        o_ref[g * gr:(g + 1) * gr, :] = (r * outm).astype(o_ref.dtype)


def kernel(x, w3, b3, w1, b1, w2, b2, wr, br):
    n, h, w, cin = x.shape
    cout = w3.shape[1]
    cmid = w1.shape[1]
    rt = 256 // cin

    eye = jnp.eye(rt, dtype=jnp.float32)
    wm_f = jnp.kron(eye, w3.T)
    wa_f = jnp.kron(eye, w1.T)
    wma = jnp.concatenate([wm_f, wa_f], axis=0).astype(jnp.bfloat16)
    wm = wm_f.astype(jnp.bfloat16)
    ws = jnp.kron(eye, jnp.ones((cin, 1), jnp.float32) @ w2
                  ).astype(jnp.bfloat16)
    bm = jnp.tile(b3.T, (rt, 1))
    bma = jnp.concatenate([bm, jnp.tile(b1.T, (rt, 1))], axis=0)
    wrb = jnp.tile(wr.T, (rt, 1))
    brb = jnp.tile(br.T, (rt, 1))

    x2 = jnp.transpose(x, (0, 1, 3, 2)).reshape(n * h * cin, w)

    slices = n * h
    sl_per_step = min(64, slices)
    rows = sl_per_step * cin
    grid = (slices // sl_per_step,)
    ngroups = sl_per_step // rt
    full = lambda i: (0, 0)
    y = pl.pallas_call(
        functools.partial(_sab_kernel, cin=cin, cmid=cmid, rt=rt,
                          ngroups=ngroups),
        out_shape=jax.ShapeDtypeStruct((slices * cout, w), x.dtype),
        grid_spec=pltpu.PrefetchScalarGridSpec(
            num_scalar_prefetch=0,
            grid=grid,
            in_specs=[
                pl.BlockSpec((rows, w), lambda i: (i, 0)),
                pl.BlockSpec(wma.shape, full),
                pl.BlockSpec(wm.shape, full),
                pl.BlockSpec(ws.shape, full),
                pl.BlockSpec(bma.shape, full),
                pl.BlockSpec(bm.shape, full),
                pl.BlockSpec(b2.shape, full),
                pl.BlockSpec(wrb.shape, full),
                pl.BlockSpec(brb.shape, full),
            ],
            out_specs=pl.BlockSpec((sl_per_step * cout, w), lambda i: (i, 0)),
        ),
        compiler_params=pltpu.CompilerParams(
            dimension_semantics=("parallel",),
            vmem_limit_bytes=64 * 1024 * 1024),
    )(x2, wma, wm, ws, bma, bm, b2, wrb, brb)
    return jnp.transpose(y.reshape(n, h, cout, w), (0, 1, 3, 2))


# speedup vs baseline: 1.5216x; 1.5216x over previous
import functools

import jax
import jax.numpy as jnp
from jax import lax
from jax.experimental import pallas as pl
from jax.experimental.pallas import tpu as pltpu


def _gelu_exact(x):
    return 0.5 * x * (1.0 + lax.erf(x * 0.7071067811865476))


def _sab_kernel(x_ref, wm_ref, wa_ref, ws_ref, wrk_ref, bm_ref, ba_ref,
                b2_ref, br_ref, o_ref, *, cin, cmid, rt, ngroups):
    gr = rt * cin
    wl = x_ref.shape[1]
    xb = x_ref[...].astype(jnp.bfloat16)
    xcat = jnp.concatenate(
        [xb[g * gr:(g + 1) * gr, :] for g in range(ngroups)], axis=1)
    hm = _gelu_exact(
        jnp.dot(wm_ref[...], xcat, preferred_element_type=jnp.float32)
        + bm_ref[...])
    ha = _gelu_exact(
        jnp.dot(wa_ref[...], xcat, preferred_element_type=jnp.float32)
        + ba_ref[...])
    outm = jnp.dot(wm_ref[...], hm.astype(jnp.bfloat16),
                   preferred_element_type=jnp.float32) + bm_ref[...]
    s = jax.nn.sigmoid(
        jnp.dot(ws_ref[...], ha.astype(jnp.bfloat16),
                preferred_element_type=jnp.float32) + b2_ref[...])
    r = jnp.dot(wrk_ref[...], s.astype(jnp.bfloat16),
                preferred_element_type=jnp.float32) + br_ref[...]
    y = (r * outm).astype(o_ref.dtype)
    for g in range(ngroups):
        o_ref[g * gr:(g + 1) * gr, :] = y[:, g * wl:(g + 1) * wl]


def kernel(x, w3, b3, w1, b1, w2, b2, wr, br):
    n, h, w, cin = x.shape
    cout = w3.shape[1]
    cmid = w1.shape[1]
    rt = 256 // cin

    eye = jnp.eye(rt, dtype=jnp.float32)
    wm = jnp.kron(eye, w3.T).astype(jnp.bfloat16)
    wa = jnp.kron(eye, w1.T).astype(jnp.bfloat16)
    ws = jnp.kron(eye, w2).astype(jnp.bfloat16)
    wrk = jnp.kron(eye, wr.T).astype(jnp.bfloat16)
    bm = jnp.tile(b3.T, (rt, 1))
    ba = jnp.tile(b1.T, (rt, 1))
    brb = jnp.tile(br.T, (rt, 1))

    x2 = jnp.transpose(x, (0, 1, 3, 2)).reshape(n * h * cin, w)

    slices = n * h
    sl_per_step = min(64, slices)
    rows = sl_per_step * cin
    grid = (slices // sl_per_step,)
    ngroups = sl_per_step // rt
    full = lambda i: (0, 0)
    y = pl.pallas_call(
        functools.partial(_sab_kernel, cin=cin, cmid=cmid, rt=rt,
                          ngroups=ngroups),
        out_shape=jax.ShapeDtypeStruct((slices * cout, w), x.dtype),
        grid_spec=pltpu.PrefetchScalarGridSpec(
            num_scalar_prefetch=0,
            grid=grid,
            in_specs=[
                pl.BlockSpec((rows, w), lambda i: (i, 0)),
                pl.BlockSpec(wm.shape, full),
                pl.BlockSpec(wa.shape, full),
                pl.BlockSpec(ws.shape, full),
                pl.BlockSpec(wrk.shape, full),
                pl.BlockSpec(bm.shape, full),
                pl.BlockSpec(ba.shape, full),
                pl.BlockSpec(b2.shape, full),
                pl.BlockSpec(brb.shape, full),
            ],
            out_specs=pl.BlockSpec((sl_per_step * cout, w), lambda i: (i, 0)),
        ),
        compiler_params=pltpu.CompilerParams(
            dimension_semantics=("parallel",),
            vmem_limit_bytes=64 * 1024 * 1024),
    )(x2, wm, wa, ws, wrk, bm, ba, b2, brb)
    return jnp.transpose(y.reshape(n, h, cout, w), (0, 1, 3, 2))


# speedup vs baseline: 1.8591x; 1.2218x over previous
import functools

import jax
import jax.numpy as jnp
from jax import lax
from jax.experimental import pallas as pl
from jax.experimental.pallas import tpu as pltpu


def _gelu_exact(x):
    return 0.5 * x * (1.0 + lax.erf(x * 0.7071067811865476))


def _sab_kernel(x_ref, wm_ref, wa_ref, ws_ref, wrk_ref, bm_ref, ba_ref,
                b2_ref, br_ref, o_ref, *, cin, cmid, rt, ngroups):
    gr = rt * cin
    wl = x_ref.shape[1]
    xb = x_ref[...].astype(jnp.bfloat16)
    xcat = jnp.concatenate(
        [xb[g * gr:(g + 1) * gr, :] for g in range(ngroups)], axis=1)
    hm = _gelu_exact(
        jnp.dot(wm_ref[...], xcat, preferred_element_type=jnp.float32)
        + bm_ref[...])
    ha = _gelu_exact(
        jnp.dot(wa_ref[...], xcat, preferred_element_type=jnp.float32)
        + ba_ref[...])
    outm = jnp.dot(wm_ref[...], hm.astype(jnp.bfloat16),
                   preferred_element_type=jnp.float32) + bm_ref[...]
    s = jax.nn.sigmoid(
        jnp.dot(ws_ref[...], ha.astype(jnp.bfloat16),
                preferred_element_type=jnp.float32) + b2_ref[...])
    r = jnp.dot(wrk_ref[...], s.astype(jnp.bfloat16),
                preferred_element_type=jnp.float32) + br_ref[...]
    y = (r * outm).astype(o_ref.dtype)
    for g in range(ngroups):
        o_ref[g * gr:(g + 1) * gr, :] = y[:, g * wl:(g + 1) * wl]


def kernel(x, w3, b3, w1, b1, w2, b2, wr, br):
    n, h, w, cin = x.shape
    cout = w3.shape[1]
    cmid = w1.shape[1]
    rt = 256 // cin

    eye = jnp.eye(rt, dtype=jnp.float32)
    wm = jnp.kron(eye, w3.T).astype(jnp.bfloat16)
    wa = jnp.kron(eye, w1.T).astype(jnp.bfloat16)
    ws = jnp.kron(eye, w2).astype(jnp.bfloat16)
    wrk = jnp.kron(eye, wr.T).astype(jnp.bfloat16)
    bm = jnp.tile(b3.T, (rt, 1))
    ba = jnp.tile(b1.T, (rt, 1))
    brb = jnp.tile(br.T, (rt, 1))

    x2 = jnp.transpose(x, (0, 1, 3, 2)).reshape(n * h * cin, w)

    slices = n * h
    sl_per_step = min(128, slices)
    rows = sl_per_step * cin
    grid = (slices // sl_per_step,)
    ngroups = sl_per_step // rt
    full = lambda i: (0, 0)
    y = pl.pallas_call(
        functools.partial(_sab_kernel, cin=cin, cmid=cmid, rt=rt,
                          ngroups=ngroups),
        out_shape=jax.ShapeDtypeStruct((slices * cout, w), x.dtype),
        grid_spec=pltpu.PrefetchScalarGridSpec(
            num_scalar_prefetch=0,
            grid=grid,
            in_specs=[
                pl.BlockSpec((rows, w), lambda i: (i, 0)),
                pl.BlockSpec(wm.shape, full),
                pl.BlockSpec(wa.shape, full),
                pl.BlockSpec(ws.shape, full),
                pl.BlockSpec(wrk.shape, full),
                pl.BlockSpec(bm.shape, full),
                pl.BlockSpec(ba.shape, full),
                pl.BlockSpec(b2.shape, full),
                pl.BlockSpec(brb.shape, full),
            ],
            out_specs=pl.BlockSpec((sl_per_step * cout, w), lambda i: (i, 0)),
        ),
        compiler_params=pltpu.CompilerParams(
            dimension_semantics=("parallel",),
            vmem_limit_bytes=64 * 1024 * 1024),
    )(x2, wm, wa, ws, wrk, bm, ba, b2, brb)
    return jnp.transpose(y.reshape(n, h, cout, w), (0, 1, 3, 2))


# speedup vs baseline: 2.0689x; 1.1129x over previous
import functools

import jax
import jax.numpy as jnp
from jax import lax
from jax.experimental import pallas as pl
from jax.experimental.pallas import tpu as pltpu


def _gelu_exact(x):
    return 0.5 * x * (1.0 + lax.erf(x * 0.7071067811865476))


def _sab_kernel(x_ref, wm_ref, wa_ref, ws_ref, wrk_ref, bm_ref, ba_ref,
                b2_ref, br_ref, o_ref, *, cin, cmid, rt, ngroups):
    gr = rt * cin
    wl = x_ref.shape[1]
    xb = x_ref[...].astype(jnp.bfloat16)
    xcat = jnp.concatenate(
        [xb[g * gr:(g + 1) * gr, :] for g in range(ngroups)], axis=1)
    hm = _gelu_exact(
        jnp.dot(wm_ref[...], xcat, preferred_element_type=jnp.float32)
        + bm_ref[...])
    ha = _gelu_exact(
        jnp.dot(wa_ref[...], xcat, preferred_element_type=jnp.float32)
        + ba_ref[...])
    outm = jnp.dot(wm_ref[...], hm.astype(jnp.bfloat16),
                   preferred_element_type=jnp.float32) + bm_ref[...]
    s = jax.nn.sigmoid(
        jnp.dot(ws_ref[...], ha.astype(jnp.bfloat16),
                preferred_element_type=jnp.float32) + b2_ref[...])
    r = jnp.dot(wrk_ref[...], s.astype(jnp.bfloat16),
                preferred_element_type=jnp.float32) + br_ref[...]
    y = (r * outm).astype(o_ref.dtype)
    for g in range(ngroups):
        o_ref[g * gr:(g + 1) * gr, :] = y[:, g * wl:(g + 1) * wl]


def kernel(x, w3, b3, w1, b1, w2, b2, wr, br):
    n, h, w, cin = x.shape
    cout = w3.shape[1]
    cmid = w1.shape[1]
    rt = 256 // cin

    eye = jnp.eye(rt, dtype=jnp.float32)
    wm = jnp.kron(eye, w3.T).astype(jnp.bfloat16)
    wa = jnp.kron(eye, w1.T).astype(jnp.bfloat16)
    ws = jnp.kron(eye, w2).astype(jnp.bfloat16)
    wrk = jnp.kron(eye, wr.T).astype(jnp.bfloat16)
    bm = jnp.tile(b3.T, (rt, 1))
    ba = jnp.tile(b1.T, (rt, 1))
    brb = jnp.tile(br.T, (rt, 1))

    x2 = jnp.transpose(x, (0, 1, 3, 2)).reshape(n * h * cin, w)

    slices = n * h
    sl_per_step = min(256, slices)
    rows = sl_per_step * cin
    grid = (slices // sl_per_step,)
    ngroups = sl_per_step // rt
    full = lambda i: (0, 0)
    y = pl.pallas_call(
        functools.partial(_sab_kernel, cin=cin, cmid=cmid, rt=rt,
                          ngroups=ngroups),
        out_shape=jax.ShapeDtypeStruct((slices * cout, w), x.dtype),
        grid_spec=pltpu.PrefetchScalarGridSpec(
            num_scalar_prefetch=0,
            grid=grid,
            in_specs=[
                pl.BlockSpec((rows, w), lambda i: (i, 0)),
                pl.BlockSpec(wm.shape, full),
                pl.BlockSpec(wa.shape, full),
                pl.BlockSpec(ws.shape, full),
                pl.BlockSpec(wrk.shape, full),
                pl.BlockSpec(bm.shape, full),
                pl.BlockSpec(ba.shape, full),
                pl.BlockSpec(b2.shape, full),
                pl.BlockSpec(brb.shape, full),
            ],
            out_specs=pl.BlockSpec((sl_per_step * cout, w), lambda i: (i, 0)),
        ),
        compiler_params=pltpu.CompilerParams(
            dimension_semantics=("parallel",),
            vmem_limit_bytes=64 * 1024 * 1024),
    )(x2, wm, wa, ws, wrk, bm, ba, b2, brb)
    return jnp.transpose(y.reshape(n, h, cout, w), (0, 1, 3, 2))


# speedup vs baseline: 2.0992x; 1.0147x over previous
import functools

import jax
import jax.numpy as jnp
from jax import lax
from jax.experimental import pallas as pl
from jax.experimental.pallas import tpu as pltpu


def _gelu_exact(x):
    return 0.5 * x * (1.0 + lax.erf(x * 0.7071067811865476))


def _sab_kernel(x_ref, wm_ref, wa_ref, ws_ref, wrk_ref, bm_ref, ba_ref,
                b2_ref, br_ref, o_ref, *, cin, cmid, rt, ngroups):
    gr = rt * cin
    wl = x_ref.shape[1]
    xb = x_ref[...].astype(jnp.bfloat16)
    xcat = jnp.concatenate(
        [xb[g * gr:(g + 1) * gr, :] for g in range(ngroups)], axis=1)
    hm = _gelu_exact(
        jnp.dot(wm_ref[...], xcat, preferred_element_type=jnp.float32)
        + bm_ref[...])
    ha = _gelu_exact(
        jnp.dot(wa_ref[...], xcat, preferred_element_type=jnp.float32)
        + ba_ref[...])
    outm = jnp.dot(wm_ref[...], hm.astype(jnp.bfloat16),
                   preferred_element_type=jnp.float32) + bm_ref[...]
    s = jax.nn.sigmoid(
        jnp.dot(ws_ref[...], ha.astype(jnp.bfloat16),
                preferred_element_type=jnp.float32) + b2_ref[...])
    r = jnp.dot(wrk_ref[...], s.astype(jnp.bfloat16),
                preferred_element_type=jnp.float32) + br_ref[...]
    y = (r * outm).astype(o_ref.dtype)
    for g in range(ngroups):
        o_ref[g * gr:(g + 1) * gr, :] = y[:, g * wl:(g + 1) * wl]


def kernel(x, w3, b3, w1, b1, w2, b2, wr, br):
    n, h, w, cin = x.shape
    cout = w3.shape[1]
    cmid = w1.shape[1]
    rt = 256 // cin

    eye = jnp.eye(rt, dtype=jnp.float32)
    wm = jnp.kron(eye, w3.T).astype(jnp.bfloat16)
    wa = jnp.kron(eye, w1.T).astype(jnp.bfloat16)
    ws = jnp.kron(eye, w2).astype(jnp.bfloat16)
    wrk = jnp.kron(eye, wr.T).astype(jnp.bfloat16)
    bm = jnp.tile(b3.T, (rt, 1))
    ba = jnp.tile(b1.T, (rt, 1))
    brb = jnp.tile(br.T, (rt, 1))

    x2 = jnp.transpose(x, (0, 1, 3, 2)).reshape(n * h * cin, w)

    slices = n * h
    sl_per_step = min(512, slices)
    rows = sl_per_step * cin
    grid = (slices // sl_per_step,)
    ngroups = sl_per_step // rt
    full = lambda i: (0, 0)
    y = pl.pallas_call(
        functools.partial(_sab_kernel, cin=cin, cmid=cmid, rt=rt,
                          ngroups=ngroups),
        out_shape=jax.ShapeDtypeStruct((slices * cout, w), x.dtype),
        grid_spec=pltpu.PrefetchScalarGridSpec(
            num_scalar_prefetch=0,
            grid=grid,
            in_specs=[
                pl.BlockSpec((rows, w), lambda i: (i, 0)),
                pl.BlockSpec(wm.shape, full),
                pl.BlockSpec(wa.shape, full),
                pl.BlockSpec(ws.shape, full),
                pl.BlockSpec(wrk.shape, full),
                pl.BlockSpec(bm.shape, full),
                pl.BlockSpec(ba.shape, full),
                pl.BlockSpec(b2.shape, full),
                pl.BlockSpec(brb.shape, full),
            ],
            out_specs=pl.BlockSpec((sl_per_step * cout, w), lambda i: (i, 0)),
        ),
        compiler_params=pltpu.CompilerParams(
            dimension_semantics=("parallel",),
            vmem_limit_bytes=64 * 1024 * 1024),
    )(x2, wm, wa, ws, wrk, bm, ba, b2, brb)
    return jnp.transpose(y.reshape(n, h, cout, w), (0, 1, 3, 2))


# speedup vs baseline: 2.1894x; 1.0430x over previous
import functools

import jax
import jax.numpy as jnp
from jax import lax
from jax.experimental import pallas as pl
from jax.experimental.pallas import tpu as pltpu


def _gelu_exact(x):
    return 0.5 * x * (1.0 + lax.erf(x * 0.7071067811865476))


def _sab_kernel(x_ref, wm_ref, wa_ref, ws_ref, wrk_ref, bm_ref, ba_ref,
                b2_ref, br_ref, o_ref, *, cin, cmid, rt, ngroups):
    gr = rt * cin
    wl = x_ref.shape[1]
    xb = x_ref[...].astype(jnp.bfloat16)
    xcat = jnp.concatenate(
        [xb[g * gr:(g + 1) * gr, :] for g in range(ngroups)], axis=1)
    hm = _gelu_exact(
        (jnp.dot(wm_ref[...], xcat, preferred_element_type=jnp.float32)
         + bm_ref[...]).astype(jnp.bfloat16))
    ha = _gelu_exact(
        (jnp.dot(wa_ref[...], xcat, preferred_element_type=jnp.float32)
         + ba_ref[...]).astype(jnp.bfloat16))
    outm = jnp.dot(wm_ref[...], hm,
                   preferred_element_type=jnp.float32) + bm_ref[...]
    s = jax.nn.sigmoid(
        jnp.dot(ws_ref[...], ha,
                preferred_element_type=jnp.float32) + b2_ref[...])
    r = jnp.dot(wrk_ref[...], s.astype(jnp.bfloat16),
                preferred_element_type=jnp.float32) + br_ref[...]
    y = (r * outm).astype(o_ref.dtype)
    for g in range(ngroups):
        o_ref[g * gr:(g + 1) * gr, :] = y[:, g * wl:(g + 1) * wl]


def kernel(x, w3, b3, w1, b1, w2, b2, wr, br):
    n, h, w, cin = x.shape
    cout = w3.shape[1]
    cmid = w1.shape[1]
    rt = 256 // cin

    eye = jnp.eye(rt, dtype=jnp.float32)
    wm = jnp.kron(eye, w3.T).astype(jnp.bfloat16)
    wa = jnp.kron(eye, w1.T).astype(jnp.bfloat16)
    ws = jnp.kron(eye, w2).astype(jnp.bfloat16)
    wrk = jnp.kron(eye, wr.T).astype(jnp.bfloat16)
    bm = jnp.tile(b3.T, (rt, 1))
    ba = jnp.tile(b1.T, (rt, 1))
    brb = jnp.tile(br.T, (rt, 1))

    x2 = jnp.transpose(x, (0, 1, 3, 2)).reshape(n * h * cin, w)

    slices = n * h
    sl_per_step = min(512, slices)
    rows = sl_per_step * cin
    grid = (slices // sl_per_step,)
    ngroups = sl_per_step // rt
    full = lambda i: (0, 0)
    y = pl.pallas_call(
        functools.partial(_sab_kernel, cin=cin, cmid=cmid, rt=rt,
                          ngroups=ngroups),
        out_shape=jax.ShapeDtypeStruct((slices * cout, w), x.dtype),
        grid_spec=pltpu.PrefetchScalarGridSpec(
            num_scalar_prefetch=0,
            grid=grid,
            in_specs=[
                pl.BlockSpec((rows, w), lambda i: (i, 0)),
                pl.BlockSpec(wm.shape, full),
                pl.BlockSpec(wa.shape, full),
                pl.BlockSpec(ws.shape, full),
                pl.BlockSpec(wrk.shape, full),
                pl.BlockSpec(bm.shape, full),
                pl.BlockSpec(ba.shape, full),
                pl.BlockSpec(b2.shape, full),
                pl.BlockSpec(brb.shape, full),
            ],
            out_specs=pl.BlockSpec((sl_per_step * cout, w), lambda i: (i, 0)),
        ),
        compiler_params=pltpu.CompilerParams(
            dimension_semantics=("parallel",),
            vmem_limit_bytes=64 * 1024 * 1024),
    )(x2, wm, wa, ws, wrk, bm, ba, b2, brb)
    return jnp.transpose(y.reshape(n, h, cout, w), (0, 1, 3, 2))
